# Initial kernel scaffold; baseline (speedup 1.0000x reference)
import jax
import jax.numpy as jnp
from jax.experimental import pallas as pl


def kernel(inputs, targets, features_memory):
    raise NotImplementedError("write your pallas kernel here")


# speedup vs baseline: 5.3001x; 5.3001x over previous
import jax
import jax.numpy as jnp
from jax import lax
from jax.experimental import pallas as pl
from jax.experimental.pallas import tpu as pltpu
from jax.experimental.pallas import tpu_sc as plsc

F = 128
C = 100000
B = 1024
MOM = 0.01
L = 16
NW = 32
ROWS_PER = C // NW


def _body(x_hbm, t_hbm, mem_hbm, out_hbm, t_v, xrow, grow, orow):
    wid = lax.axis_index("s") * 2 + lax.axis_index("c")
    row0 = wid * ROWS_PER
    base = row0 * F

    pltpu.sync_copy(mem_hbm.at[pl.ds(base, ROWS_PER * F)],
                    out_hbm.at[pl.ds(base, ROWS_PER * F)])
    pltpu.sync_copy(t_hbm, t_v.at[pl.ds(0, B)])

    def step(i, carry):
        tv = t_v[pl.ds(i, L)][0]
        local = tv - row0

        @pl.when((local >= 0) & (local < ROWS_PER))
        def _():
            pltpu.sync_copy(out_hbm.at[pl.ds(tv * F, F)], grow)
            pltpu.sync_copy(x_hbm.at[pl.ds(i * F, F)], xrow)
            acc = jnp.zeros((L,), jnp.float32)
            for k in range(F // L):
                g = grow[pl.ds(k * L, L)]
                xv = xrow[pl.ds(k * L, L)]
                u = MOM * g + (1.0 - MOM) * xv
                orow[pl.ds(k * L, L)] = u
                acc = acc + u * u
            s = jnp.full((L,), jnp.sum(acc), jnp.float32)
            y = plsc.bitcast(
                jnp.int32(0x5F3759DF) - (plsc.bitcast(s, jnp.int32) >> 1),
                jnp.float32)
            for _ in range(3):
                y = y * (1.5 - 0.5 * s * y * y)
            for k in range(F // L):
                orow[pl.ds(k * L, L)] = orow[pl.ds(k * L, L)] * y
            pltpu.sync_copy(orow, out_hbm.at[pl.ds(tv * F, F)])

        return carry

    lax.fori_loop(0, B, step, 0)


@jax.jit
def kernel(inputs, targets, features_memory):
    x_flat = inputs.reshape(B * F)
    mem_flat = features_memory.reshape(C * F)
    t32 = targets.astype(jnp.int32)

    mesh = plsc.VectorSubcoreMesh(core_axis_name="c", subcore_axis_name="s")
    out_flat = pl.kernel(
        _body,
        out_type=jax.ShapeDtypeStruct((C * F,), jnp.float32),
        mesh=mesh,
        compiler_params=pltpu.CompilerParams(needs_layout_passes=False),
        scratch_types=[
            pltpu.VMEM((B + L,), jnp.int32),
            pltpu.VMEM((F,), jnp.float32),
            pltpu.VMEM((F,), jnp.float32),
            pltpu.VMEM((F,), jnp.float32),
        ],
    )(x_flat, t32, mem_flat)
    return out_flat.reshape(C, F)


# speedup vs baseline: 73.9310x; 13.9491x over previous
import jax
import jax.numpy as jnp
from jax import lax
from jax.experimental import pallas as pl
from jax.experimental.pallas import tpu as pltpu
from jax.experimental.pallas import tpu_sc as plsc

F = 128
C = 100000
B = 1024
MOM = 0.01
L = 16
NW = 32
ROWS_PER = C // NW


def _body(x_hbm, t_hbm, out_hbm, t_v, xrow, grow, orow):
    wid = lax.axis_index("s") * 2 + lax.axis_index("c")
    row0 = wid * ROWS_PER

    pltpu.sync_copy(t_hbm, t_v.at[pl.ds(0, B)])

    def step(i, carry):
        tv = t_v[pl.ds(i, L)][0]
        local = tv - row0

        @pl.when((local >= 0) & (local < ROWS_PER))
        def _():
            pltpu.sync_copy(out_hbm.at[pl.ds(tv * F, F)], grow)
            pltpu.sync_copy(x_hbm.at[pl.ds(i * F, F)], xrow)
            acc = jnp.zeros((L,), jnp.float32)
            for k in range(F // L):
                g = grow[pl.ds(k * L, L)]
                xv = xrow[pl.ds(k * L, L)]
                u = MOM * g + (1.0 - MOM) * xv
                orow[pl.ds(k * L, L)] = u
                acc = acc + u * u
            s = jnp.full((L,), jnp.sum(acc), jnp.float32)
            y = plsc.bitcast(
                jnp.int32(0x5F3759DF) - (plsc.bitcast(s, jnp.int32) >> 1),
                jnp.float32)
            for _ in range(3):
                y = y * (1.5 - 0.5 * s * y * y)
            for k in range(F // L):
                orow[pl.ds(k * L, L)] = orow[pl.ds(k * L, L)] * y
            pltpu.sync_copy(orow, out_hbm.at[pl.ds(tv * F, F)])

        return carry

    lax.fori_loop(0, B, step, 0)


@jax.jit
def kernel(inputs, targets, features_memory):
    x_flat = inputs.reshape(B * F)
    t32 = targets.astype(jnp.int32)

    mem_ref = jax.new_ref(features_memory.reshape(C * F))

    mesh = plsc.VectorSubcoreMesh(core_axis_name="c", subcore_axis_name="s")
    pl.kernel(
        _body,
        out_type=(),
        mesh=mesh,
        compiler_params=pltpu.CompilerParams(needs_layout_passes=False),
        scratch_types=[
            pltpu.VMEM((B + L,), jnp.int32),
            pltpu.VMEM((F,), jnp.float32),
            pltpu.VMEM((F,), jnp.float32),
            pltpu.VMEM((F,), jnp.float32),
        ],
    )(x_flat, t32, mem_ref)
    return mem_ref[...].reshape(C, F)


# speedup vs baseline: 97.8978x; 1.3242x over previous
import jax
import jax.numpy as jnp
from jax import lax
from jax.experimental import pallas as pl
from jax.experimental.pallas import tpu as pltpu
from jax.experimental.pallas import tpu_sc as plsc

F = 128
C = 100000
B = 1024
MOM = 0.01
L = 16
NW = 32
ROWS_PER = C // NW

_GATHER_DNUMS = lax.GatherDimensionNumbers(
    offset_dims=(), collapsed_slice_dims=(0,), start_index_map=(0,))


def _compute_rows(xg, gg, og):
    for j in range(L):
        acc = jnp.zeros((L,), jnp.float32)
        for k in range(F // L):
            g = gg[j, pl.ds(k * L, L)]
            xv = xg[j, pl.ds(k * L, L)]
            u = MOM * g + (1.0 - MOM) * xv
            og[j, pl.ds(k * L, L)] = u
            acc = acc + u * u
        s = jnp.full((L,), jnp.sum(acc), jnp.float32)
        y = plsc.bitcast(
            jnp.int32(0x5F3759DF) - (plsc.bitcast(s, jnp.int32) >> 1),
            jnp.float32)
        for _ in range(3):
            y = y * (1.5 - 0.5 * s * y * y)
        for k in range(F // L):
            og[j, pl.ds(k * L, L)] = og[j, pl.ds(k * L, L)] * y


def _body(x_hbm, t_hbm, mem_hbm, t_v, tloc, iloc, xg, gg, og, sem1, sem2):
    wid = lax.axis_index("s") * 2 + lax.axis_index("c")
    row0 = wid * ROWS_PER
    iota = lax.iota(jnp.int32, L)

    pltpu.sync_copy(t_hbm, t_v.at[pl.ds(0, B)])

    def scan_chunk(c, off):
        tvec = t_v[pl.ds(c * L, L)]
        local = tvec - row0
        msk = (local >= 0) & (local < ROWS_PER)
        plsc.store_compressed(tloc.at[pl.ds(off, L)], tvec, mask=msk)
        plsc.store_compressed(iloc.at[pl.ds(off, L)], c * L + iota, mask=msk)
        return off + plsc.all_reduce_population_count(msk)[0]

    n = lax.fori_loop(0, B // L, scan_chunk, jnp.int32(0))

    @pl.when(n > 0)
    def _():
        tlast = tloc[pl.ds(n - 1, L)][0]
        ilast = iloc[pl.ds(n - 1, L)][0]
        tloc[pl.ds(n, L)] = jnp.full((L,), tlast, jnp.int32)
        iloc[pl.ds(n, L)] = jnp.full((L,), ilast, jnp.int32)
        ngroups = (n + L - 1) // L

        def group(g, carry):
            base = g * L
            treg = tloc[pl.ds(base, L)]
            ireg = iloc[pl.ds(base, L)]
            cnt = jnp.minimum(n - base, L)
            cp1 = pltpu.async_copy(x_hbm.at[ireg], xg, sem1)
            cp2 = pltpu.async_copy(mem_hbm.at[treg], gg, sem2)
            cp1.wait()
            cp2.wait()
            tchk = jnp.where(iota < cnt, treg, -1 - iota)
            skey, _ = plsc.sort_key_val(tchk, tchk)
            rolled = lax.gather(
                skey, ((iota + 1) & (L - 1))[:, None], _GATHER_DNUMS,
                slice_sizes=(1,),
                mode=lax.GatherScatterMode.PROMISE_IN_BOUNDS)
            ndup = plsc.all_reduce_population_count(
                (skey == rolled) & (iota < L - 1))[0]

            @pl.when(ndup == 0)
            def _fast():
                _compute_rows(xg, gg, og)
                pltpu.async_copy(og, mem_hbm.at[treg], sem1).wait()

            @pl.when(ndup > 0)
            def _slow():
                def persample(j, c2):
                    tv = tloc[pl.ds(base + j, L)][0]
                    iv = iloc[pl.ds(base + j, L)][0]
                    tvv = jnp.full((L,), tv, jnp.int32)
                    ivv = jnp.full((L,), iv, jnp.int32)
                    c1 = pltpu.async_copy(x_hbm.at[ivv], xg, sem1)
                    c2_ = pltpu.async_copy(mem_hbm.at[tvv], gg, sem2)
                    c1.wait()
                    c2_.wait()
                    _compute_rows(xg, gg, og)
                    pltpu.async_copy(og, mem_hbm.at[tvv], sem1).wait()
                    return c2

                lax.fori_loop(0, cnt, persample, 0)

            return carry

        lax.fori_loop(0, ngroups, group, jnp.int32(0))


@jax.jit
def kernel(inputs, targets, features_memory):
    t32 = targets.astype(jnp.int32)

    mem_ref = jax.new_ref(features_memory)

    mesh = plsc.VectorSubcoreMesh(core_axis_name="c", subcore_axis_name="s")
    pl.kernel(
        _body,
        out_type=(),
        mesh=mesh,
        compiler_params=pltpu.CompilerParams(needs_layout_passes=False),
        scratch_types=[
            pltpu.VMEM((B + L,), jnp.int32),
            pltpu.VMEM((B + 2 * L,), jnp.int32),
            pltpu.VMEM((B + 2 * L,), jnp.int32),
            pltpu.VMEM((L, F), jnp.float32),
            pltpu.VMEM((L, F), jnp.float32),
            pltpu.VMEM((L, F), jnp.float32),
            pltpu.SemaphoreType.DMA,
            pltpu.SemaphoreType.DMA,
        ],
    )(inputs, t32, mem_ref)
    return mem_ref[...]


# speedup vs baseline: 142.6907x; 1.4575x over previous
import jax
import jax.numpy as jnp
from jax import lax
from jax.experimental import pallas as pl
from jax.experimental.pallas import tpu as pltpu
from jax.experimental.pallas import tpu_sc as plsc

F = 128
C = 100000
B = 1024
MOM = 0.01
L = 16
NW = 32
ROWS_PER = C // NW

_GATHER_DNUMS = lax.GatherDimensionNumbers(
    offset_dims=(), collapsed_slice_dims=(0,), start_index_map=(0,))


def _compute_rows(xg, gg, og):
    for j in range(L):
        acc = jnp.zeros((L,), jnp.float32)
        for k in range(F // L):
            g = gg[j, pl.ds(k * L, L)]
            xv = xg[j, pl.ds(k * L, L)]
            u = MOM * g + (1.0 - MOM) * xv
            og[j, pl.ds(k * L, L)] = u
            acc = acc + u * u
        s = jnp.full((L,), jnp.sum(acc), jnp.float32)
        y = plsc.bitcast(
            jnp.int32(0x5F3759DF) - (plsc.bitcast(s, jnp.int32) >> 1),
            jnp.float32)
        for _ in range(3):
            y = y * (1.5 - 0.5 * s * y * y)
        for k in range(F // L):
            og[j, pl.ds(k * L, L)] = og[j, pl.ds(k * L, L)] * y


def _body(x_hbm, t_hbm, mem_hbm, t_v, tloc, iloc, xg, gg, og, sem1, sem2):
    wid = lax.axis_index("s") * 2 + lax.axis_index("c")
    row0 = wid * ROWS_PER
    iota = lax.iota(jnp.int32, L)

    pltpu.sync_copy(t_hbm, t_v.at[pl.ds(0, B)])

    def scan_chunk(c, off):
        tvec = t_v[pl.ds(c * L, L)]
        local = tvec - row0
        msk = (local >= 0) & (local < ROWS_PER)
        plsc.store_compressed(tloc.at[pl.ds(off, L)], tvec, mask=msk)
        plsc.store_compressed(iloc.at[pl.ds(off, L)], c * L + iota, mask=msk)
        return off + plsc.all_reduce_population_count(msk)[0]

    n = lax.fori_loop(0, B // L, scan_chunk, jnp.int32(0))

    @pl.when(n > 0)
    def _():
        tlast = tloc[pl.ds(n - 1, L)][0]
        ilast = iloc[pl.ds(n - 1, L)][0]
        tloc[pl.ds(n, L)] = jnp.full((L,), tlast, jnp.int32)
        iloc[pl.ds(n, L)] = jnp.full((L,), ilast, jnp.int32)
        ngroups = (n + L - 1) // L

        def group(g, carry):
            base = g * L
            treg = tloc[pl.ds(base, L)]
            ireg = iloc[pl.ds(base, L)]
            cnt = jnp.minimum(n - base, L)
            cp1 = pltpu.async_copy(x_hbm.at[ireg], xg, sem1)
            cp2 = pltpu.async_copy(mem_hbm.at[treg], gg, sem2)
            cp1.wait()
            cp2.wait()
            tchk = jnp.where(iota < cnt, treg, -1 - iota)
            skey, _ = plsc.sort_key_val(tchk, tchk)
            rolled = lax.gather(
                skey, ((iota + 1) & (L - 1))[:, None], _GATHER_DNUMS,
                slice_sizes=(1,),
                mode=lax.GatherScatterMode.PROMISE_IN_BOUNDS)
            ndup = plsc.all_reduce_population_count(
                (skey == rolled) & (iota < L - 1))[0]

            @pl.when(ndup == 0)
            def _fast():
                _compute_rows(xg, gg, og)
                pltpu.async_copy(og, mem_hbm.at[treg], sem1).wait()

            @pl.when(ndup > 0)
            def _slow():
                for j in range(L):
                    tj = treg[j]
                    acc = jnp.zeros((L,), jnp.float32)
                    for k in range(F // L):
                        b = gg[j, pl.ds(k * L, L)]
                        for k2 in range(j):
                            b = jnp.where(treg[k2] == tj,
                                          og[k2, pl.ds(k * L, L)], b)
                        u = MOM * b + (1.0 - MOM) * xg[j, pl.ds(k * L, L)]
                        og[j, pl.ds(k * L, L)] = u
                        acc = acc + u * u
                    s = jnp.full((L,), jnp.sum(acc), jnp.float32)
                    y = plsc.bitcast(
                        jnp.int32(0x5F3759DF)
                        - (plsc.bitcast(s, jnp.int32) >> 1), jnp.float32)
                    for _ in range(3):
                        y = y * (1.5 - 0.5 * s * y * y)
                    for k in range(F // L):
                        og[j, pl.ds(k * L, L)] = og[j, pl.ds(k * L, L)] * y
                for j in range(L):
                    @pl.when(j < cnt)
                    def _scatter_one(j=j):
                        pltpu.sync_copy(og.at[pl.ds(j, 1), :],
                                        mem_hbm.at[pl.ds(treg[j], 1), :])

            return carry

        lax.fori_loop(0, ngroups, group, jnp.int32(0))


@jax.jit
def kernel(inputs, targets, features_memory):
    t32 = targets.astype(jnp.int32)

    mem_ref = jax.new_ref(features_memory)

    mesh = plsc.VectorSubcoreMesh(core_axis_name="c", subcore_axis_name="s")
    pl.kernel(
        _body,
        out_type=(),
        mesh=mesh,
        compiler_params=pltpu.CompilerParams(needs_layout_passes=False),
        scratch_types=[
            pltpu.VMEM((B + L,), jnp.int32),
            pltpu.VMEM((B + 2 * L,), jnp.int32),
            pltpu.VMEM((B + 2 * L,), jnp.int32),
            pltpu.VMEM((L, F), jnp.float32),
            pltpu.VMEM((L, F), jnp.float32),
            pltpu.VMEM((L, F), jnp.float32),
            pltpu.SemaphoreType.DMA,
            pltpu.SemaphoreType.DMA,
        ],
    )(inputs, t32, mem_ref)
    return mem_ref[...]


# speedup vs baseline: 143.0483x; 1.0025x over previous
import jax
import jax.numpy as jnp
from jax import lax
from jax.experimental import pallas as pl
from jax.experimental.pallas import tpu as pltpu
from jax.experimental.pallas import tpu_sc as plsc

F = 128
C = 100000
B = 1024
MOM = 0.01
L = 16
NW = 32
ROWS_PER = C // NW

_GATHER_DNUMS = lax.GatherDimensionNumbers(
    offset_dims=(), collapsed_slice_dims=(0,), start_index_map=(0,))


def _compute_rows(xg, gg, og):
    for j in range(L):
        acc = jnp.zeros((L,), jnp.float32)
        for k in range(F // L):
            g = gg[j, pl.ds(k * L, L)]
            xv = xg[j, pl.ds(k * L, L)]
            u = MOM * g + (1.0 - MOM) * xv
            og[j, pl.ds(k * L, L)] = u
            acc = acc + u * u
        s = jnp.full((L,), jnp.sum(acc), jnp.float32)
        y = plsc.bitcast(
            jnp.int32(0x5F3759DF) - (plsc.bitcast(s, jnp.int32) >> 1),
            jnp.float32)
        for _ in range(3):
            y = y * (1.5 - 0.5 * s * y * y)
        for k in range(F // L):
            og[j, pl.ds(k * L, L)] = og[j, pl.ds(k * L, L)] * y


def _body(x_hbm, t_hbm, mem_hbm, t_v, tloc, iloc, xg, gg, og,
          sem1, sem2, sem3):
    wid = lax.axis_index("s") * 2 + lax.axis_index("c")
    row0 = wid * ROWS_PER
    iota = lax.iota(jnp.int32, L)

    pltpu.sync_copy(t_hbm, t_v.at[pl.ds(0, B)])

    def scan_chunk(c, off):
        tvec = t_v[pl.ds(c * L, L)]
        local = tvec - row0
        msk = (local >= 0) & (local < ROWS_PER)
        plsc.store_compressed(tloc.at[pl.ds(off, L)], tvec, mask=msk)
        plsc.store_compressed(iloc.at[pl.ds(off, L)], c * L + iota, mask=msk)
        return off + plsc.all_reduce_population_count(msk)[0]

    n = lax.fori_loop(0, B // L, scan_chunk, jnp.int32(0), unroll=4)

    @pl.when(n > 0)
    def _():
        tlast = tloc[pl.ds(n - 1, L)][0]
        ilast = iloc[pl.ds(n - 1, L)][0]
        tloc[pl.ds(n, L)] = jnp.full((L,), tlast, jnp.int32)
        iloc[pl.ds(n, L)] = jnp.full((L,), ilast, jnp.int32)
        ngroups = (n + L - 1) // L

        def group(g, carry):
            pending, tprev = carry
            base = g * L
            treg = tloc[pl.ds(base, L)]
            ireg = iloc[pl.ds(base, L)]
            cnt = jnp.minimum(n - base, L)

            ovacc = jnp.zeros((L,), jnp.bool_)
            for r in range(L):
                ovacc = ovacc | (treg == tprev[r])
            hazard = plsc.all_reduce_population_count(ovacc)[0] > 0
            drain = pltpu.make_async_copy(mem_hbm.at[treg], og, sem3)

            @pl.when((pending == 1) & hazard)
            def _():
                drain.wait()

            cp1 = pltpu.async_copy(x_hbm.at[ireg], xg, sem1)
            cp2 = pltpu.async_copy(mem_hbm.at[treg], gg, sem2)
            cp1.wait()
            cp2.wait()

            @pl.when((pending == 1) & jnp.logical_not(hazard))
            def _():
                drain.wait()
            tchk = jnp.where(iota < cnt, treg, -1 - iota)
            skey, _ = plsc.sort_key_val(tchk, tchk)
            rolled = lax.gather(
                skey, ((iota + 1) & (L - 1))[:, None], _GATHER_DNUMS,
                slice_sizes=(1,),
                mode=lax.GatherScatterMode.PROMISE_IN_BOUNDS)
            ndup = plsc.all_reduce_population_count(
                (skey == rolled) & (iota < L - 1))[0]

            @pl.when(ndup == 0)
            def _fast():
                _compute_rows(xg, gg, og)
                pltpu.async_copy(og, mem_hbm.at[treg], sem3)

            @pl.when(ndup > 0)
            def _slow():
                for j in range(L):
                    tj = treg[j]
                    acc = jnp.zeros((L,), jnp.float32)
                    for k in range(F // L):
                        b = gg[j, pl.ds(k * L, L)]
                        for k2 in range(j):
                            b = jnp.where(treg[k2] == tj,
                                          og[k2, pl.ds(k * L, L)], b)
                        u = MOM * b + (1.0 - MOM) * xg[j, pl.ds(k * L, L)]
                        og[j, pl.ds(k * L, L)] = u
                        acc = acc + u * u
                    s = jnp.full((L,), jnp.sum(acc), jnp.float32)
                    y = plsc.bitcast(
                        jnp.int32(0x5F3759DF)
                        - (plsc.bitcast(s, jnp.int32) >> 1), jnp.float32)
                    for _ in range(3):
                        y = y * (1.5 - 0.5 * s * y * y)
                    for k in range(F // L):
                        og[j, pl.ds(k * L, L)] = og[j, pl.ds(k * L, L)] * y
                for j in range(L):
                    @pl.when(j < cnt)
                    def _scatter_one(j=j):
                        pltpu.sync_copy(og.at[pl.ds(j, 1), :],
                                        mem_hbm.at[pl.ds(treg[j], 1), :])

            return (jnp.where(ndup == 0, jnp.int32(1), jnp.int32(0)), treg)

        pending, tprev = lax.fori_loop(
            0, ngroups, group,
            (jnp.int32(0), jnp.full((L,), -1, jnp.int32)))

        @pl.when(pending == 1)
        def _():
            pltpu.make_async_copy(mem_hbm.at[tprev], og, sem3).wait()


@jax.jit
def kernel(inputs, targets, features_memory):
    t32 = targets.astype(jnp.int32)

    mem_ref = jax.new_ref(features_memory)

    mesh = plsc.VectorSubcoreMesh(core_axis_name="c", subcore_axis_name="s")
    pl.kernel(
        _body,
        out_type=(),
        mesh=mesh,
        compiler_params=pltpu.CompilerParams(needs_layout_passes=False),
        scratch_types=[
            pltpu.VMEM((B + L,), jnp.int32),
            pltpu.VMEM((B + 2 * L,), jnp.int32),
            pltpu.VMEM((B + 2 * L,), jnp.int32),
            pltpu.VMEM((L, F), jnp.float32),
            pltpu.VMEM((L, F), jnp.float32),
            pltpu.VMEM((L, F), jnp.float32),
            pltpu.SemaphoreType.DMA,
            pltpu.SemaphoreType.DMA,
            pltpu.SemaphoreType.DMA,
        ],
    )(inputs, t32, mem_ref)
    return mem_ref[...]
